# baseline (speedup 1.0000x reference)
import functools

import jax
import jax.numpy as jnp
from jax import lax
from jax.experimental import pallas as pl
from jax.experimental.pallas import tpu as pltpu
from jax.experimental.pallas import tpu_sc as plsc

B, D, V = 16384, 128, 1000000

NC = 2
NS = 16
NW = NC * NS
BPW = B // NW
CH = 128
K = BPW // CH


def _gather_body(idx_hbm, table_hbm, out_hbm, idx_v, vals_v, sem):
    wid = lax.axis_index("s") * NC + lax.axis_index("c")
    pltpu.sync_copy(idx_hbm.at[pl.ds(wid * K, K)], idx_v)
    copies = [
        pltpu.async_copy(table_hbm.at[idx_v.at[j]], vals_v.at[j], sem)
        for j in range(K)
    ]
    for c in copies:
        c.wait()
    pltpu.sync_copy(vals_v, out_hbm.at[pl.ds(wid * K, K)])


_gather = pl.kernel(
    _gather_body,
    mesh=plsc.VectorSubcoreMesh(core_axis_name="c", subcore_axis_name="s"),
    out_type=jax.ShapeDtypeStruct((NW * K, CH), jnp.float32),
    scratch_types=[
        pltpu.VMEM((K, CH), jnp.int32),
        pltpu.VMEM((K, CH), jnp.float32),
        pltpu.SemaphoreType.DMA,
    ],
)

BM = 1024
GRID = B // BM
_INV = 1.0 / float(B * D)


def _loss_body(l_ref, r_ref, a_ref, out_ref):
    i = pl.program_id(0)

    @pl.when(i == 0)
    def _init():
        out_ref[0, 0] = 0.0

    part = jnp.sum(jnp.abs(l_ref[...] - a_ref[...] * r_ref[...]))
    out_ref[0, 0] += part

    @pl.when(i == GRID - 1)
    def _finish():
        out_ref[0, 0] = out_ref[0, 0] * _INV


_loss = pl.pallas_call(
    _loss_body,
    grid=(GRID,),
    in_specs=[
        pl.BlockSpec((BM, D), lambda i: (i, 0)),
        pl.BlockSpec((BM, D), lambda i: (i, 0)),
        pl.BlockSpec((BM, 1), lambda i: (i, 0)),
    ],
    out_specs=pl.BlockSpec(memory_space=pltpu.SMEM),
    out_shape=jax.ShapeDtypeStruct((1, 1), jnp.float32),
    compiler_params=pltpu.CompilerParams(dimension_semantics=("arbitrary",)),
)


def kernel(emd_l, emd_r, player, alpha_weight):
    idx = player.astype(jnp.int32).reshape(NW * K, CH)
    table = alpha_weight.reshape(V)
    a = _gather(idx, table).reshape(B, 1)
    return _loss(emd_l, emd_r, a)[0, 0]


# speedup vs baseline: 1.1140x; 1.1140x over previous
import jax
import jax.numpy as jnp
from jax import lax
from jax.experimental import pallas as pl
from jax.experimental.pallas import tpu as pltpu
from jax.experimental.pallas import tpu_sc as plsc

B, D, V = 16384, 128, 1000000

NC = 2
NS = 16
NL = 16
NW = NC * NS
BPW = B // NW
CH = 128
K = BPW // CH
NBUF = 2
_INV = 1.0 / float(B * D)


def _sc_body(idx_hbm, table_hbm, l_hbm, r_hbm, out_hbm,
             idx_v, alpha_v, lbuf, rbuf, acc_v, sem_a, sem_d):
    wid = lax.axis_index("s") * NC + lax.axis_index("c")
    base = wid * BPW

    pltpu.sync_copy(idx_hbm.at[pl.ds(wid * K, K)], idx_v)
    a_cps = [
        pltpu.async_copy(table_hbm.at[idx_v.at[j]], alpha_v.at[j], sem_a)
        for j in range(K)
    ]

    d_cps = {}

    def fire(c):
        s = c % NBUF
        d_cps[c] = (
            pltpu.async_copy(l_hbm.at[pl.ds(base + c * CH, CH), :], lbuf.at[s], sem_d),
            pltpu.async_copy(r_hbm.at[pl.ds(base + c * CH, CH), :], rbuf.at[s], sem_d),
        )

    fire(0)
    fire(1)
    for cp in a_cps:
        cp.wait()

    acc = jnp.zeros((NL,), jnp.float32)
    for c in range(K):
        for cp in d_cps[c]:
            cp.wait()
        s = c % NBUF

        def group_body(g, a, s=s, c=c):
            a16 = alpha_v[c, pl.ds(g * NL, NL)]
            for j in range(NL):
                a_s = a16[j]
                r = g * NL + j
                for gg in range(D // NL):
                    lv = lbuf[s, r, pl.ds(gg * NL, NL)]
                    rv = rbuf[s, r, pl.ds(gg * NL, NL)]
                    a = a + jnp.abs(lv - a_s * rv)
            return a

        acc = lax.fori_loop(0, CH // NL, group_body, acc)
        if c + NBUF < K:
            fire(c + NBUF)

    acc_v[...] = acc
    pltpu.sync_copy(acc_v, out_hbm.at[wid])


_sc_loss = pl.kernel(
    _sc_body,
    mesh=plsc.VectorSubcoreMesh(core_axis_name="c", subcore_axis_name="s"),
    out_type=jax.ShapeDtypeStruct((NW, NL), jnp.float32),
    scratch_types=[
        pltpu.VMEM((K, CH), jnp.int32),
        pltpu.VMEM((K, CH), jnp.float32),
        pltpu.VMEM((NBUF, CH, D), jnp.float32),
        pltpu.VMEM((NBUF, CH, D), jnp.float32),
        pltpu.VMEM((NL,), jnp.float32),
        pltpu.SemaphoreType.DMA,
        pltpu.SemaphoreType.DMA,
    ],
)


def _fin_body(p_ref, out_ref):
    out_ref[0, 0] = jnp.sum(p_ref[...]) * _INV


_finish = pl.pallas_call(
    _fin_body,
    out_specs=pl.BlockSpec(memory_space=pltpu.SMEM),
    out_shape=jax.ShapeDtypeStruct((1, 1), jnp.float32),
)


def kernel(emd_l, emd_r, player, alpha_weight):
    idx = player.astype(jnp.int32).reshape(NW * K, CH)
    table = alpha_weight.reshape(V)
    parts = _sc_loss(idx, table, emd_l, emd_r)
    return _finish(parts)[0, 0]
